# baseline (speedup 1.0000x reference)
import functools

import jax
import jax.numpy as jnp
from jax import lax
from jax.experimental import pallas as pl
from jax.experimental.pallas import tpu as pltpu
from jax.experimental.pallas import tpu_sc as plsc

_N = 10000
_E = 320000
_H = 128
_NS = 16
_ROWS_A = 632
_ROWS_LAST = _N - 15 * _ROWS_A
_EDGES_PER_TILE = _E // _NS
_CHUNK = 128
_NCHUNK = 160
_EPAD = _NCHUNK * _CHUNK
_PHASES = 4
_PCH = _NCHUNK // _PHASES


def _sc_body(x_hbm, s0_hbm, d0_hbm, s1_hbm, d1_hbm, out_hbm, acc,
             si, di, rb0, rb1, g0, g1, s0sem, s1sem):
    c = lax.axis_index("c")
    s = lax.axis_index("s")
    rb = [rb0, rb1]
    gsem = [g0, g1]
    ssem = [s0sem, s1sem]

    r0 = s * _ROWS_A

    @pl.when(s < _NS - 1)
    def _():
        pltpu.sync_copy(x_hbm.at[pl.ds(r0, _ROWS_A)],
                        acc.at[pl.ds(r0, _ROWS_A)])

    @pl.when(s == _NS - 1)
    def _():
        pltpu.sync_copy(x_hbm.at[pl.ds(15 * _ROWS_A, _ROWS_LAST)],
                        acc.at[pl.ds(15 * _ROWS_A, _ROWS_LAST)])

    plsc.subcore_barrier()

    def run_direction(src_hbm, dst_hbm):
        def issue_gather(j, b):
            pltpu.async_copy(x_hbm.at[si.at[j]], rb[b], gsem[b])

        def wait_gather(b):
            pltpu.make_async_copy(x_hbm.at[pl.ds(0, _CHUNK)], rb[b],
                                  gsem[b]).wait()

        def issue_scatter(j, b):
            pltpu.async_copy(rb[b], acc.at[di.at[j]], ssem[b], add=True)

        def wait_scatter(b):
            pltpu.make_async_copy(rb[b], acc.at[pl.ds(0, _CHUNK)],
                                  ssem[b]).wait()

        def phase_body(ph, carry):
            base = s * _NCHUNK + ph * _PCH
            pltpu.sync_copy(src_hbm.at[pl.ds(base, _PCH)], si)
            pltpu.sync_copy(dst_hbm.at[pl.ds(base, _PCH)], di)
            issue_gather(0, 0)
            issue_gather(1, 1)
            wait_gather(0)
            issue_scatter(0, 0)

            def pair_body(k, carry2):
                j1 = 2 * k + 1
                wait_gather(1)
                issue_scatter(j1, 1)
                wait_scatter(0)
                issue_gather(j1 + 1, 0)
                j2 = 2 * k + 2
                wait_gather(0)
                issue_scatter(j2, 0)
                wait_scatter(1)
                issue_gather(j2 + 1, 1)
                return carry2

            lax.fori_loop(0, (_PCH - 2) // 2, pair_body, 0)
            wait_gather(1)
            issue_scatter(_PCH - 1, 1)
            wait_scatter(0)
            wait_scatter(1)
            return carry

        lax.fori_loop(0, _PHASES, phase_body, 0)

    @pl.when(c == 0)
    def _():
        run_direction(s0_hbm, d0_hbm)

    @pl.when(c == 1)
    def _():
        run_direction(s1_hbm, d1_hbm)

    plsc.subcore_barrier()

    @pl.when(s < _NS - 1)
    def _():
        pltpu.sync_copy(acc.at[pl.ds(r0, _ROWS_A)],
                        out_hbm.at[c, pl.ds(r0, _ROWS_A)])

    @pl.when(s == _NS - 1)
    def _():
        pltpu.sync_copy(acc.at[pl.ds(15 * _ROWS_A, _ROWS_LAST)],
                        out_hbm.at[c, pl.ds(15 * _ROWS_A, _ROWS_LAST)])


_sc_agg = functools.partial(
    pl.kernel,
    out_type=jax.ShapeDtypeStruct((2, _N, _H), jnp.float32),
    mesh=plsc.VectorSubcoreMesh(core_axis_name="c", subcore_axis_name="s"),
    scratch_types=[
        pltpu.VMEM_SHARED((_N, _H), jnp.float32),
        pltpu.VMEM((_PCH, _CHUNK), jnp.int32),
        pltpu.VMEM((_PCH, _CHUNK), jnp.int32),
        pltpu.VMEM((_CHUNK, _H), jnp.float32),
        pltpu.VMEM((_CHUNK, _H), jnp.float32),
        pltpu.SemaphoreType.DMA,
        pltpu.SemaphoreType.DMA,
        pltpu.SemaphoreType.DMA,
        pltpu.SemaphoreType.DMA,
    ],
)(_sc_body)


_BLK = 1000


def _dense_body(h1_ref, h2_ref, W1t_ref, b1_ref, g1_ref, be1_ref,
                W2t_ref, b2_ref, g2_ref, be2_ref,
                Wl1a_ref, Wl1b_ref, bl1_ref, Wl2t_ref, bl2_ref, out_ref):
    def resblock(h, Wt, b, g, be):
        z = jnp.dot(h, Wt, preferred_element_type=jnp.float32) + b
        mu = jnp.mean(z, axis=-1, keepdims=True)
        var = jnp.mean((z - mu) * (z - mu), axis=-1, keepdims=True)
        ln = (z - mu) * lax.rsqrt(var + 1e-5) * g + be
        return h + jnp.maximum(ln, 0.0)

    r1 = resblock(h1_ref[:], W1t_ref[:], b1_ref[:], g1_ref[:], be1_ref[:])
    r2 = resblock(h2_ref[:], W2t_ref[:], b2_ref[:], g2_ref[:], be2_ref[:])
    hmid = jnp.maximum(
        jnp.dot(r1, Wl1a_ref[:], preferred_element_type=jnp.float32)
        + jnp.dot(r2, Wl1b_ref[:], preferred_element_type=jnp.float32)
        + bl1_ref[:], 0.0)
    out_ref[:] = jnp.maximum(
        jnp.dot(hmid, Wl2t_ref[:], preferred_element_type=jnp.float32)
        + bl2_ref[:], 0.0)


def _row_spec(nrows, ncols):
    return pl.BlockSpec((nrows, ncols), lambda i: (i, 0))


def _full_spec(nrows, ncols):
    return pl.BlockSpec((nrows, ncols), lambda i: (0, 0))


_dense_call = pl.pallas_call(
    _dense_body,
    grid=(_N // _BLK,),
    in_specs=[
        _row_spec(_BLK, _H), _row_spec(_BLK, _H),
        _full_spec(_H, _H), _full_spec(1, _H), _full_spec(1, _H), _full_spec(1, _H),
        _full_spec(_H, _H), _full_spec(1, _H), _full_spec(1, _H), _full_spec(1, _H),
        _full_spec(_H, 2 * _H), _full_spec(_H, 2 * _H), _full_spec(1, 2 * _H),
        _full_spec(2 * _H, _H), _full_spec(1, _H),
    ],
    out_specs=_row_spec(_BLK, _H),
    out_shape=jax.ShapeDtypeStruct((_N, _H), jnp.float32),
)


def _pad_chunks(e, fill):
    per_tile = e.reshape(_NS, _EDGES_PER_TILE)
    pad = jnp.full((_NS, _EPAD - _EDGES_PER_TILE), fill, jnp.int32)
    return jnp.concatenate([per_tile, pad], axis=1).reshape(
        _NS * _NCHUNK, _CHUNK)


@jax.jit
def _impl(x, ei, W1, b1, g1, be1, W2, b2, g2, be2, Wl1, bl1, Wl2, bl2):
    x_pad = jnp.concatenate([x, jnp.zeros((8, _H), jnp.float32)], axis=0)
    s0 = _pad_chunks(ei[0], _N)
    d0 = _pad_chunks(ei[1], 0)
    s1 = _pad_chunks(ei[1], _N)
    d1 = _pad_chunks(ei[0], 0)
    h12 = _sc_agg(x_pad, s0, d0, s1, d1)
    return _dense_call(
        h12[0], h12[1],
        W1.T, b1[None, :], g1[None, :], be1[None, :],
        W2.T, b2[None, :], g2[None, :], be2[None, :],
        Wl1.T[:_H], Wl1.T[_H:], bl1[None, :],
        Wl2.T, bl2[None, :],
    )


def kernel(x, ei, W1, b1, g1, be1, W2, b2, g2, be2, Wl1, bl1, Wl2, bl2):
    return _impl(x, ei, W1, b1, g1, be1, W2, b2, g2, be2, Wl1, bl1, Wl2, bl2)


# speedup vs baseline: 2.2913x; 2.2913x over previous
import functools

import jax
import jax.numpy as jnp
from jax import lax
from jax.experimental import pallas as pl
from jax.experimental.pallas import tpu as pltpu
from jax.experimental.pallas import tpu_sc as plsc

_N = 10000
_E = 320000
_H = 128
_NS = 16
_ROWS_A = 632
_ROWS_LAST = _N - 15 * _ROWS_A
_EDGES_PER_TILE = _E // _NS
_CHUNK = 128
_NCHUNK = 160
_EPAD = _NCHUNK * _CHUNK
_PHASES = 4
_PCH = _NCHUNK // _PHASES


def _sc_body(x_hbm, s0_hbm, d0_hbm, s1_hbm, d1_hbm, out_hbm, acc,
             si, di, rb0, rb1, g0, g1, s0sem, s1sem):
    c = lax.axis_index("c")
    s = lax.axis_index("s")
    rb = [rb0, rb1]
    gsem = [g0, g1]
    ssem = [s0sem, s1sem]

    r0 = s * _ROWS_A

    @pl.when(s < _NS - 1)
    def _():
        pltpu.sync_copy(x_hbm.at[pl.ds(r0, _ROWS_A)],
                        acc.at[pl.ds(r0, _ROWS_A)])

    @pl.when(s == _NS - 1)
    def _():
        pltpu.sync_copy(x_hbm.at[pl.ds(15 * _ROWS_A, _ROWS_LAST)],
                        acc.at[pl.ds(15 * _ROWS_A, _ROWS_LAST)])

    plsc.subcore_barrier()

    def run_direction(src_hbm, dst_hbm):
        def issue_gather(j, b):
            pltpu.async_copy(x_hbm.at[si.at[j]], rb[b], gsem[b])

        def wait_gather(b):
            pltpu.make_async_copy(x_hbm.at[pl.ds(0, _CHUNK)], rb[b],
                                  gsem[b]).wait()

        def issue_scatter(j, b):
            pltpu.async_copy(rb[b], acc.at[di.at[j]], ssem[b], add=True)

        def wait_scatter(b):
            pltpu.make_async_copy(rb[b], acc.at[pl.ds(0, _CHUNK)],
                                  ssem[b]).wait()

        def phase_body(ph, carry):
            base = s * _NCHUNK + ph * _PCH
            pltpu.sync_copy(src_hbm.at[pl.ds(base, _PCH)], si)
            pltpu.sync_copy(dst_hbm.at[pl.ds(base, _PCH)], di)
            issue_gather(0, 0)
            issue_gather(1, 1)
            wait_gather(0)
            issue_scatter(0, 0)

            def pair_body(k, carry2):
                j1 = 2 * k + 1
                wait_gather(1)
                issue_scatter(j1, 1)
                wait_scatter(0)
                issue_gather(j1 + 1, 0)
                j2 = 2 * k + 2
                wait_gather(0)
                issue_scatter(j2, 0)
                wait_scatter(1)
                issue_gather(j2 + 1, 1)
                return carry2

            lax.fori_loop(0, (_PCH - 2) // 2, pair_body, 0)
            wait_gather(1)
            issue_scatter(_PCH - 1, 1)
            wait_scatter(0)
            wait_scatter(1)
            return carry

        lax.fori_loop(0, _PHASES, phase_body, 0)

    @pl.when(c == 0)
    def _():
        run_direction(s0_hbm, d0_hbm)

    @pl.when(c == 1)
    def _():
        run_direction(s1_hbm, d1_hbm)

    plsc.subcore_barrier()

    @pl.when(s < _NS - 1)
    def _():
        pltpu.sync_copy(acc.at[pl.ds(r0, _ROWS_A)],
                        out_hbm.at[c, pl.ds(r0, _ROWS_A)])

    @pl.when(s == _NS - 1)
    def _():
        pltpu.sync_copy(acc.at[pl.ds(15 * _ROWS_A, _ROWS_LAST)],
                        out_hbm.at[c, pl.ds(15 * _ROWS_A, _ROWS_LAST)])


_sc_agg = functools.partial(
    pl.kernel,
    out_type=jax.ShapeDtypeStruct((2, _N, _H), jnp.float32),
    mesh=plsc.VectorSubcoreMesh(core_axis_name="c", subcore_axis_name="s"),
    scratch_types=[
        pltpu.VMEM_SHARED((_N + _NS, _H), jnp.float32),
        pltpu.VMEM((_PCH, _CHUNK), jnp.int32),
        pltpu.VMEM((_PCH, _CHUNK), jnp.int32),
        pltpu.VMEM((_CHUNK, _H), jnp.float32),
        pltpu.VMEM((_CHUNK, _H), jnp.float32),
        pltpu.SemaphoreType.DMA,
        pltpu.SemaphoreType.DMA,
        pltpu.SemaphoreType.DMA,
        pltpu.SemaphoreType.DMA,
    ],
)(_sc_body)


_BLK = 1000


def _dense_body(h1_ref, h2_ref, W1t_ref, b1_ref, g1_ref, be1_ref,
                W2t_ref, b2_ref, g2_ref, be2_ref,
                Wl1a_ref, Wl1b_ref, bl1_ref, Wl2t_ref, bl2_ref, out_ref):
    def resblock(h, Wt, b, g, be):
        z = jnp.dot(h, Wt, preferred_element_type=jnp.float32) + b
        mu = jnp.mean(z, axis=-1, keepdims=True)
        var = jnp.mean((z - mu) * (z - mu), axis=-1, keepdims=True)
        ln = (z - mu) * lax.rsqrt(var + 1e-5) * g + be
        return h + jnp.maximum(ln, 0.0)

    r1 = resblock(h1_ref[:], W1t_ref[:], b1_ref[:], g1_ref[:], be1_ref[:])
    r2 = resblock(h2_ref[:], W2t_ref[:], b2_ref[:], g2_ref[:], be2_ref[:])
    hmid = jnp.maximum(
        jnp.dot(r1, Wl1a_ref[:], preferred_element_type=jnp.float32)
        + jnp.dot(r2, Wl1b_ref[:], preferred_element_type=jnp.float32)
        + bl1_ref[:], 0.0)
    out_ref[:] = jnp.maximum(
        jnp.dot(hmid, Wl2t_ref[:], preferred_element_type=jnp.float32)
        + bl2_ref[:], 0.0)


def _row_spec(nrows, ncols):
    return pl.BlockSpec((nrows, ncols), lambda i: (i, 0))


def _full_spec(nrows, ncols):
    return pl.BlockSpec((nrows, ncols), lambda i: (0, 0))


_dense_call = pl.pallas_call(
    _dense_body,
    grid=(_N // _BLK,),
    in_specs=[
        _row_spec(_BLK, _H), _row_spec(_BLK, _H),
        _full_spec(_H, _H), _full_spec(1, _H), _full_spec(1, _H), _full_spec(1, _H),
        _full_spec(_H, _H), _full_spec(1, _H), _full_spec(1, _H), _full_spec(1, _H),
        _full_spec(_H, 2 * _H), _full_spec(_H, 2 * _H), _full_spec(1, 2 * _H),
        _full_spec(2 * _H, _H), _full_spec(1, _H),
    ],
    out_specs=_row_spec(_BLK, _H),
    out_shape=jax.ShapeDtypeStruct((_N, _H), jnp.float32),
)


def _pad_chunks(e, fill):
    per_tile = e.reshape(_NS, _EDGES_PER_TILE)
    pad = jnp.broadcast_to(fill[:, None], (_NS, _EPAD - _EDGES_PER_TILE))
    return jnp.concatenate([per_tile, pad], axis=1).reshape(
        _NS * _NCHUNK, _CHUNK)


@jax.jit
def _impl(x, ei, W1, b1, g1, be1, W2, b2, g2, be2, Wl1, bl1, Wl2, bl2):
    x_pad = jnp.concatenate([x, jnp.zeros((8, _H), jnp.float32)], axis=0)
    srcfill = _N + (jnp.arange(_NS, dtype=jnp.int32) % 8)
    dstfill = _N + jnp.arange(_NS, dtype=jnp.int32)
    s0 = _pad_chunks(ei[0], srcfill)
    d0 = _pad_chunks(ei[1], dstfill)
    s1 = _pad_chunks(ei[1], srcfill)
    d1 = _pad_chunks(ei[0], dstfill)
    h12 = _sc_agg(x_pad, s0, d0, s1, d1)
    return _dense_call(
        h12[0], h12[1],
        W1.T, b1[None, :], g1[None, :], be1[None, :],
        W2.T, b2[None, :], g2[None, :], be2[None, :],
        Wl1.T[:_H], Wl1.T[_H:], bl1[None, :],
        Wl2.T, bl2[None, :],
    )


def kernel(x, ei, W1, b1, g1, be1, W2, b2, g2, be2, Wl1, bl1, Wl2, bl2):
    return _impl(x, ei, W1, b1, g1, be1, W2, b2, g2, be2, Wl1, bl1, Wl2, bl2)


# speedup vs baseline: 3.9940x; 1.7432x over previous
import functools

import jax
import jax.numpy as jnp
from jax import lax
from jax.experimental import pallas as pl
from jax.experimental.pallas import tpu as pltpu
from jax.experimental.pallas import tpu_sc as plsc

_N = 10000
_E = 320000
_H = 128
_NS = 16
_ROWS_A = 632
_ROWS_LAST = _N - 15 * _ROWS_A
_EDGES_PER_TILE = _E // _NS
_CHUNK = 128
_NFULL = _EDGES_PER_TILE // _CHUNK
_TAIL = _EDGES_PER_TILE - _NFULL * _CHUNK
_GRP = 12


def _sc_body(x_hbm, ei0_hbm, ei1_hbm, out_hbm, acc,
             si0, si1, si2, si3, di0, di1, di2, di3,
             rb0, rb1, rb2, sit, dit,
             i0, i1, i2, i3, g0, g1, g2, tsem):
    c = lax.axis_index("c")
    s = lax.axis_index("s")
    si = [si0, si1, si2, si3]
    di = [di0, di1, di2, di3]
    isem = [i0, i1, i2, i3]
    rb = [rb0, rb1, rb2]
    gsem = [g0, g1, g2]

    r0 = s * _ROWS_A

    @pl.when(s < _NS - 1)
    def _():
        pltpu.sync_copy(x_hbm.at[pl.ds(r0, _ROWS_A)],
                        acc.at[pl.ds(r0, _ROWS_A)])

    @pl.when(s == _NS - 1)
    def _():
        pltpu.sync_copy(x_hbm.at[pl.ds(15 * _ROWS_A, _ROWS_LAST)],
                        acc.at[pl.ds(15 * _ROWS_A, _ROWS_LAST)])

    plsc.subcore_barrier()

    ebase = s * _EDGES_PER_TILE

    def run_direction(src_hbm, dst_hbm):
        def issue_idx(j, b):
            q = b % 4
            off = ebase + j * _CHUNK
            pltpu.async_copy(src_hbm.at[pl.ds(off, _CHUNK)], si[q], isem[q])
            pltpu.async_copy(dst_hbm.at[pl.ds(off, _CHUNK)], di[q], isem[q])

        def wait_idx(b):
            q = b % 4
            pltpu.make_async_copy(src_hbm.at[pl.ds(ebase, _CHUNK)], si[q],
                                  isem[q]).wait()
            pltpu.make_async_copy(dst_hbm.at[pl.ds(ebase, _CHUNK)], di[q],
                                  isem[q]).wait()

        def issue_gather(b):
            pltpu.async_copy(x_hbm.at[si[b % 4]], rb[b % 3], gsem[b % 3])

        def wait_gather(b):
            pltpu.make_async_copy(x_hbm.at[pl.ds(0, _CHUNK)], rb[b % 3],
                                  gsem[b % 3]).wait()

        def scatter(b):
            pltpu.sync_copy(rb[b % 3], acc.at[di[b % 4]], add=True)

        def chunk_step(j, b, has_g2, has_i4):
            if has_g2:
                wait_idx(b + 2)
                issue_gather(b + 2)
            wait_gather(b)
            scatter(b)
            if has_i4:
                issue_idx(j + 4, b + 4)

        issue_idx(0, 0)
        issue_idx(1, 1)
        issue_idx(2, 2)
        issue_idx(3, 3)
        wait_idx(0)
        issue_gather(0)
        wait_idx(1)
        issue_gather(1)

        def group_body(k, carry):
            for b in range(_GRP):
                chunk_step(k * _GRP + b, b, True, True)
            return carry

        lax.fori_loop(0, (_NFULL // _GRP) - 1, group_body, 0)
        for j in range(_NFULL - _GRP, _NFULL):
            chunk_step(j, j % _GRP, j + 2 < _NFULL, j + 4 < _NFULL)

        toff = ebase + _NFULL * _CHUNK
        pltpu.sync_copy(src_hbm.at[pl.ds(toff, _TAIL)], sit)
        pltpu.sync_copy(dst_hbm.at[pl.ds(toff, _TAIL)], dit)
        pltpu.async_copy(x_hbm.at[sit], rb0.at[pl.ds(0, _TAIL)], tsem).wait()
        pltpu.sync_copy(rb0.at[pl.ds(0, _TAIL)], acc.at[dit], add=True)

    @pl.when(c == 0)
    def _():
        run_direction(ei0_hbm, ei1_hbm)

    @pl.when(c == 1)
    def _():
        run_direction(ei1_hbm, ei0_hbm)

    plsc.subcore_barrier()

    @pl.when(s < _NS - 1)
    def _():
        pltpu.sync_copy(acc.at[pl.ds(r0, _ROWS_A)],
                        out_hbm.at[c, pl.ds(r0, _ROWS_A)])

    @pl.when(s == _NS - 1)
    def _():
        pltpu.sync_copy(acc.at[pl.ds(15 * _ROWS_A, _ROWS_LAST)],
                        out_hbm.at[c, pl.ds(15 * _ROWS_A, _ROWS_LAST)])


_sc_agg = functools.partial(
    pl.kernel,
    out_type=jax.ShapeDtypeStruct((2, _N, _H), jnp.float32),
    mesh=plsc.VectorSubcoreMesh(core_axis_name="c", subcore_axis_name="s"),
    scratch_types=[
        pltpu.VMEM_SHARED((_N, _H), jnp.float32),
        pltpu.VMEM((_CHUNK,), jnp.int32),
        pltpu.VMEM((_CHUNK,), jnp.int32),
        pltpu.VMEM((_CHUNK,), jnp.int32),
        pltpu.VMEM((_CHUNK,), jnp.int32),
        pltpu.VMEM((_CHUNK,), jnp.int32),
        pltpu.VMEM((_CHUNK,), jnp.int32),
        pltpu.VMEM((_CHUNK,), jnp.int32),
        pltpu.VMEM((_CHUNK,), jnp.int32),
        pltpu.VMEM((_CHUNK, _H), jnp.float32),
        pltpu.VMEM((_CHUNK, _H), jnp.float32),
        pltpu.VMEM((_CHUNK, _H), jnp.float32),
        pltpu.VMEM((_TAIL,), jnp.int32),
        pltpu.VMEM((_TAIL,), jnp.int32),
        pltpu.SemaphoreType.DMA,
        pltpu.SemaphoreType.DMA,
        pltpu.SemaphoreType.DMA,
        pltpu.SemaphoreType.DMA,
        pltpu.SemaphoreType.DMA,
        pltpu.SemaphoreType.DMA,
        pltpu.SemaphoreType.DMA,
        pltpu.SemaphoreType.DMA,
    ],
)(_sc_body)


_BLK = 1000


def _dense_body(h1_ref, h2_ref, W1t_ref, b1_ref, g1_ref, be1_ref,
                W2t_ref, b2_ref, g2_ref, be2_ref,
                Wl1a_ref, Wl1b_ref, bl1_ref, Wl2t_ref, bl2_ref, out_ref):
    def resblock(h, Wt, b, g, be):
        z = jnp.dot(h, Wt, preferred_element_type=jnp.float32) + b
        mu = jnp.mean(z, axis=-1, keepdims=True)
        var = jnp.mean((z - mu) * (z - mu), axis=-1, keepdims=True)
        ln = (z - mu) * lax.rsqrt(var + 1e-5) * g + be
        return h + jnp.maximum(ln, 0.0)

    r1 = resblock(h1_ref[:], W1t_ref[:], b1_ref[:], g1_ref[:], be1_ref[:])
    r2 = resblock(h2_ref[:], W2t_ref[:], b2_ref[:], g2_ref[:], be2_ref[:])
    hmid = jnp.maximum(
        jnp.dot(r1, Wl1a_ref[:], preferred_element_type=jnp.float32)
        + jnp.dot(r2, Wl1b_ref[:], preferred_element_type=jnp.float32)
        + bl1_ref[:], 0.0)
    out_ref[:] = jnp.maximum(
        jnp.dot(hmid, Wl2t_ref[:], preferred_element_type=jnp.float32)
        + bl2_ref[:], 0.0)


def _row_spec(nrows, ncols):
    return pl.BlockSpec((nrows, ncols), lambda i: (i, 0))


def _full_spec(nrows, ncols):
    return pl.BlockSpec((nrows, ncols), lambda i: (0, 0))


_dense_call = pl.pallas_call(
    _dense_body,
    grid=(_N // _BLK,),
    in_specs=[
        _row_spec(_BLK, _H), _row_spec(_BLK, _H),
        _full_spec(_H, _H), _full_spec(1, _H), _full_spec(1, _H), _full_spec(1, _H),
        _full_spec(_H, _H), _full_spec(1, _H), _full_spec(1, _H), _full_spec(1, _H),
        _full_spec(_H, 2 * _H), _full_spec(_H, 2 * _H), _full_spec(1, 2 * _H),
        _full_spec(2 * _H, _H), _full_spec(1, _H),
    ],
    out_specs=_row_spec(_BLK, _H),
    out_shape=jax.ShapeDtypeStruct((_N, _H), jnp.float32),
)


@jax.jit
def _impl(x, ei, W1, b1, g1, be1, W2, b2, g2, be2, Wl1, bl1, Wl2, bl2):
    h12 = _sc_agg(x, ei[0], ei[1])
    return _dense_call(
        h12[0], h12[1],
        W1.T, b1[None, :], g1[None, :], be1[None, :],
        W2.T, b2[None, :], g2[None, :], be2[None, :],
        Wl1.T[:_H], Wl1.T[_H:], bl1[None, :],
        Wl2.T, bl2[None, :],
    )


def kernel(x, ei, W1, b1, g1, be1, W2, b2, g2, be2, Wl1, bl1, Wl2, bl2):
    return _impl(x, ei, W1, b1, g1, be1, W2, b2, g2, be2, Wl1, bl1, Wl2, bl2)


# speedup vs baseline: 4.1587x; 1.0412x over previous
import functools

import jax
import jax.numpy as jnp
from jax import lax
from jax.experimental import pallas as pl
from jax.experimental.pallas import tpu as pltpu
from jax.experimental.pallas import tpu_sc as plsc

_N = 10000
_E = 320000
_H = 128
_NS = 16
_ROWS_A = 640
_ROWS_LAST = _N - 15 * _ROWS_A
_EDGES_PER_TILE = _E // _NS
_CHUNK = 128
_NFULL = _EDGES_PER_TILE // _CHUNK
_TAIL = _EDGES_PER_TILE - _NFULL * _CHUNK
_GRP = 12


def _sc_body(x_hbm, ei0_hbm, ei1_hbm, out0_hbm, out1_hbm, acc,
             si0, si1, si2, si3, di0, di1, di2, di3,
             rb0, rb1, rb2, sit, dit,
             i0, i1, i2, i3, g0, g1, g2, tsem):
    c = lax.axis_index("c")
    s = lax.axis_index("s")
    si = [si0, si1, si2, si3]
    di = [di0, di1, di2, di3]
    isem = [i0, i1, i2, i3]
    rb = [rb0, rb1, rb2]
    gsem = [g0, g1, g2]

    r0 = s * _ROWS_A
    ebase = s * _EDGES_PER_TILE

    def seed_acc():
        @pl.when(s < _NS - 1)
        def _():
            pltpu.sync_copy(x_hbm.at[pl.ds(r0, _ROWS_A)],
                            acc.at[pl.ds(r0, _ROWS_A)])

        @pl.when(s == _NS - 1)
        def _():
            pltpu.sync_copy(x_hbm.at[pl.ds(15 * _ROWS_A, _ROWS_LAST)],
                            acc.at[pl.ds(15 * _ROWS_A, _ROWS_LAST)])

    def run_direction(src_hbm, dst_hbm):
        def issue_idx(j, b):
            q = b % 4
            off = ebase + j * _CHUNK
            pltpu.async_copy(src_hbm.at[pl.ds(off, _CHUNK)], si[q], isem[q])
            pltpu.async_copy(dst_hbm.at[pl.ds(off, _CHUNK)], di[q], isem[q])

        def wait_idx(b):
            q = b % 4
            pltpu.make_async_copy(src_hbm.at[pl.ds(ebase, _CHUNK)], si[q],
                                  isem[q]).wait()
            pltpu.make_async_copy(dst_hbm.at[pl.ds(ebase, _CHUNK)], di[q],
                                  isem[q]).wait()

        def issue_gather(b):
            pltpu.async_copy(x_hbm.at[si[b % 4]], rb[b % 3], gsem[b % 3])

        def wait_gather(b):
            pltpu.make_async_copy(x_hbm.at[pl.ds(0, _CHUNK)], rb[b % 3],
                                  gsem[b % 3]).wait()

        def scatter(b):
            pltpu.sync_copy(rb[b % 3], acc.at[di[b % 4]], add=True)

        def chunk_step(j, b, has_g2, has_i4):
            if has_g2:
                wait_idx(b + 2)
                issue_gather(b + 2)
            wait_gather(b)
            scatter(b)
            if has_i4:
                issue_idx(j + 4, b + 4)

        issue_idx(0, 0)
        issue_idx(1, 1)
        issue_idx(2, 2)
        issue_idx(3, 3)
        wait_idx(0)
        issue_gather(0)
        wait_idx(1)
        issue_gather(1)
        seed_acc()
        plsc.subcore_barrier()

        def group_body(k, carry):
            for b in range(_GRP):
                chunk_step(k * _GRP + b, b, True, True)
            return carry

        lax.fori_loop(0, (_NFULL // _GRP) - 1, group_body, 0)
        for j in range(_NFULL - _GRP, _NFULL):
            chunk_step(j, j % _GRP, j + 2 < _NFULL, j + 4 < _NFULL)

        toff = ebase + _NFULL * _CHUNK
        pltpu.sync_copy(src_hbm.at[pl.ds(toff, _TAIL)], sit)
        pltpu.sync_copy(dst_hbm.at[pl.ds(toff, _TAIL)], dit)
        pltpu.async_copy(x_hbm.at[sit], rb0.at[pl.ds(0, _TAIL)], tsem).wait()
        pltpu.sync_copy(rb0.at[pl.ds(0, _TAIL)], acc.at[dit], add=True)

    @pl.when(c == 0)
    def _():
        run_direction(ei0_hbm, ei1_hbm)

    @pl.when(c == 1)
    def _():
        run_direction(ei1_hbm, ei0_hbm)

    plsc.subcore_barrier()

    def writeout(out_hbm):
        @pl.when(s < _NS - 1)
        def _():
            pltpu.sync_copy(acc.at[pl.ds(r0, _ROWS_A)],
                            out_hbm.at[pl.ds(r0, _ROWS_A)])

        @pl.when(s == _NS - 1)
        def _():
            pltpu.sync_copy(acc.at[pl.ds(15 * _ROWS_A, _ROWS_LAST)],
                            out_hbm.at[pl.ds(15 * _ROWS_A, _ROWS_LAST)])

    @pl.when(c == 0)
    def _():
        writeout(out0_hbm)

    @pl.when(c == 1)
    def _():
        writeout(out1_hbm)


_sc_agg = functools.partial(
    pl.kernel,
    out_type=[jax.ShapeDtypeStruct((_N, _H), jnp.float32),
              jax.ShapeDtypeStruct((_N, _H), jnp.float32)],
    mesh=plsc.VectorSubcoreMesh(core_axis_name="c", subcore_axis_name="s"),
    scratch_types=[
        pltpu.VMEM_SHARED((_N, _H), jnp.float32),
        pltpu.VMEM((_CHUNK,), jnp.int32),
        pltpu.VMEM((_CHUNK,), jnp.int32),
        pltpu.VMEM((_CHUNK,), jnp.int32),
        pltpu.VMEM((_CHUNK,), jnp.int32),
        pltpu.VMEM((_CHUNK,), jnp.int32),
        pltpu.VMEM((_CHUNK,), jnp.int32),
        pltpu.VMEM((_CHUNK,), jnp.int32),
        pltpu.VMEM((_CHUNK,), jnp.int32),
        pltpu.VMEM((_CHUNK, _H), jnp.float32),
        pltpu.VMEM((_CHUNK, _H), jnp.float32),
        pltpu.VMEM((_CHUNK, _H), jnp.float32),
        pltpu.VMEM((_TAIL,), jnp.int32),
        pltpu.VMEM((_TAIL,), jnp.int32),
        pltpu.SemaphoreType.DMA,
        pltpu.SemaphoreType.DMA,
        pltpu.SemaphoreType.DMA,
        pltpu.SemaphoreType.DMA,
        pltpu.SemaphoreType.DMA,
        pltpu.SemaphoreType.DMA,
        pltpu.SemaphoreType.DMA,
        pltpu.SemaphoreType.DMA,
    ],
)(_sc_body)


_BLK = 1000


def _dense_body(h1_ref, h2_ref, W1t_ref, b1_ref, g1_ref, be1_ref,
                W2t_ref, b2_ref, g2_ref, be2_ref,
                Wl1a_ref, Wl1b_ref, bl1_ref, Wl2t_ref, bl2_ref, out_ref):
    def resblock(h, Wt, b, g, be):
        z = jnp.dot(h, Wt, preferred_element_type=jnp.float32) + b
        mu = jnp.mean(z, axis=-1, keepdims=True)
        var = jnp.mean((z - mu) * (z - mu), axis=-1, keepdims=True)
        ln = (z - mu) * lax.rsqrt(var + 1e-5) * g + be
        return h + jnp.maximum(ln, 0.0)

    r1 = resblock(h1_ref[:], W1t_ref[:], b1_ref[:], g1_ref[:], be1_ref[:])
    r2 = resblock(h2_ref[:], W2t_ref[:], b2_ref[:], g2_ref[:], be2_ref[:])
    hmid = jnp.maximum(
        jnp.dot(r1, Wl1a_ref[:], preferred_element_type=jnp.float32)
        + jnp.dot(r2, Wl1b_ref[:], preferred_element_type=jnp.float32)
        + bl1_ref[:], 0.0)
    out_ref[:] = jnp.maximum(
        jnp.dot(hmid, Wl2t_ref[:], preferred_element_type=jnp.float32)
        + bl2_ref[:], 0.0)


def _row_spec(nrows, ncols):
    return pl.BlockSpec((nrows, ncols), lambda i: (i, 0))


def _full_spec(nrows, ncols):
    return pl.BlockSpec((nrows, ncols), lambda i: (0, 0))


_dense_call = pl.pallas_call(
    _dense_body,
    grid=(_N // _BLK,),
    in_specs=[
        _row_spec(_BLK, _H), _row_spec(_BLK, _H),
        _full_spec(_H, _H), _full_spec(1, _H), _full_spec(1, _H), _full_spec(1, _H),
        _full_spec(_H, _H), _full_spec(1, _H), _full_spec(1, _H), _full_spec(1, _H),
        _full_spec(_H, 2 * _H), _full_spec(_H, 2 * _H), _full_spec(1, 2 * _H),
        _full_spec(2 * _H, _H), _full_spec(1, _H),
    ],
    out_specs=_row_spec(_BLK, _H),
    out_shape=jax.ShapeDtypeStruct((_N, _H), jnp.float32),
)


@jax.jit
def _impl(x, ei, W1, b1, g1, be1, W2, b2, g2, be2, Wl1, bl1, Wl2, bl2):
    h1, h2 = _sc_agg(x, ei[0], ei[1])
    return _dense_call(
        h1, h2,
        W1.T, b1[None, :], g1[None, :], be1[None, :],
        W2.T, b2[None, :], g2[None, :], be2[None, :],
        Wl1.T[:_H], Wl1.T[_H:], bl1[None, :],
        Wl2.T, bl2[None, :],
    )


def kernel(x, ei, W1, b1, g1, be1, W2, b2, g2, be2, Wl1, bl1, Wl2, bl2):
    return _impl(x, ei, W1, b1, g1, be1, W2, b2, g2, be2, Wl1, bl1, Wl2, bl2)
